# Initial kernel scaffold; baseline (speedup 1.0000x reference)
import jax
import jax.numpy as jnp
from jax.experimental import pallas as pl


def kernel(x):
    raise NotImplementedError("write your pallas kernel here")


# speedup vs baseline: 11.0636x; 11.0636x over previous
import functools

import jax
import jax.numpy as jnp
from jax.experimental import pallas as pl

_K = 512


def _topk_mask_kernel(x_ref, o_ref, *, k):
    x = x_ref[:, :]
    b = x.view(jnp.int32)
    m = jnp.where(b < 0, b ^ jnp.int32(0x7FFFFFFF), b)

    cnt = jnp.sum((m >= 0).astype(jnp.int32), axis=1, keepdims=True)
    t = jnp.where(cnt >= k, jnp.int32(0), jnp.int32(-(2**31)))
    for bit in range(30, -1, -1):
        t_try = t + jnp.int32(1 << bit)
        cnt = jnp.sum((m >= t_try).astype(jnp.int32), axis=1, keepdims=True)
        t = jnp.where(cnt >= k, t_try, t)

    t_eff = jnp.maximum(t, jnp.int32(1))
    o_ref[:, :] = jnp.where(m >= t_eff, x, jnp.float32(0.0))


def kernel(x):
    rows, cols = x.shape
    block_rows = 8
    grid = (rows // block_rows,)
    return pl.pallas_call(
        functools.partial(_topk_mask_kernel, k=_K),
        grid=grid,
        in_specs=[pl.BlockSpec((block_rows, cols), lambda i: (i, 0))],
        out_specs=pl.BlockSpec((block_rows, cols), lambda i: (i, 0)),
        out_shape=jax.ShapeDtypeStruct((rows, cols), x.dtype),
    )(x)


# speedup vs baseline: 26.5512x; 2.3999x over previous
import functools

import jax
import jax.numpy as jnp
from jax.experimental import pallas as pl

_K = 512


def _count_ge(m, t, nsplit):
    cols = m.shape[1]
    w = cols // nsplit
    parts = [
        jnp.sum((m[:, i * w:(i + 1) * w] >= t).astype(jnp.int32), axis=1,
                keepdims=True)
        for i in range(nsplit)
    ]
    tot = parts[0]
    for p in parts[1:]:
        tot = tot + p
    return tot


def _topk_mask_kernel(x_ref, o_ref, *, k, nsplit):
    x = x_ref[:, :]
    b = x.view(jnp.int32)
    m = jnp.where(b < 0, b ^ jnp.int32(0x7FFFFFFF), b)
    cols = m.shape[1]

    cnt0 = _count_ge(m, jnp.zeros((m.shape[0], 1), jnp.int32), nsplit)
    sel = cnt0 >= k
    t = jnp.where(sel, jnp.int32(0), jnp.int32(-(2**31)))
    cur = jnp.where(sel, cnt0, jnp.int32(cols))

    def cond(carry):
        bit, _, cur = carry
        return jnp.logical_and(bit >= 0, jnp.max(cur) > k)

    def body(carry):
        bit, t, cur = carry
        t_try = t + jnp.left_shift(jnp.int32(1), bit)
        cnt = _count_ge(m, t_try, nsplit)
        sel = cnt >= k
        return (bit - 1, jnp.where(sel, t_try, t), jnp.where(sel, cnt, cur))

    _, t, _ = jax.lax.while_loop(cond, body, (jnp.int32(30), t, cur))

    vk = jnp.min(jnp.where(m >= t, m, jnp.int32(2**31 - 1)), axis=1,
                 keepdims=True)
    t_eff = jnp.maximum(vk, jnp.int32(1))
    o_ref[:, :] = jnp.where(m >= t_eff, x, jnp.float32(0.0))


def kernel(x):
    rows, cols = x.shape
    block_rows = 16
    grid = (rows // block_rows,)
    return pl.pallas_call(
        functools.partial(_topk_mask_kernel, k=_K, nsplit=4),
        grid=grid,
        in_specs=[pl.BlockSpec((block_rows, cols), lambda i: (i, 0))],
        out_specs=pl.BlockSpec((block_rows, cols), lambda i: (i, 0)),
        out_shape=jax.ShapeDtypeStruct((rows, cols), x.dtype),
    )(x)


# speedup vs baseline: 29.5775x; 1.1140x over previous
import functools

import jax
import jax.numpy as jnp
from jax.experimental import pallas as pl

_K = 512


def _count_ge(m, t, nsplit):
    cols = m.shape[1]
    w = cols // nsplit
    parts = [
        jnp.sum((m[:, i * w:(i + 1) * w] >= t).astype(jnp.int32), axis=1,
                keepdims=True)
        for i in range(nsplit)
    ]
    tot = parts[0]
    for p in parts[1:]:
        tot = tot + p
    return tot


def _topk_mask_kernel(x_ref, o_ref, *, k, nsplit):
    x = x_ref[:, :]
    b = x.view(jnp.int32)
    m = jnp.where(b < 0, b ^ jnp.int32(0x7FFFFFFF), b)
    cols = m.shape[1]

    cnt0 = _count_ge(m, jnp.zeros((m.shape[0], 1), jnp.int32), nsplit)
    sel = cnt0 >= k
    t = jnp.where(sel, jnp.int32(0), jnp.int32(-(2**31)))
    cur = jnp.where(sel, cnt0, jnp.int32(cols))

    def cond(carry):
        bit, _, cur = carry
        return jnp.logical_and(bit >= 0, jnp.max(cur) > k)

    def body(carry):
        bit, t, cur = carry
        t_try = t + jnp.left_shift(jnp.int32(1), bit)
        cnt = _count_ge(m, t_try, nsplit)
        sel = cnt >= k
        return (bit - 1, jnp.where(sel, t_try, t), jnp.where(sel, cnt, cur))

    _, t, _ = jax.lax.while_loop(cond, body, (jnp.int32(30), t, cur))

    vk = jnp.min(jnp.where(m >= t, m, jnp.int32(2**31 - 1)), axis=1,
                 keepdims=True)
    t_eff = jnp.maximum(vk, jnp.int32(1))
    o_ref[:, :] = jnp.where(m >= t_eff, x, jnp.float32(0.0))


def kernel(x):
    rows, cols = x.shape
    block_rows = 32
    grid = (rows // block_rows,)
    return pl.pallas_call(
        functools.partial(_topk_mask_kernel, k=_K, nsplit=8),
        grid=grid,
        in_specs=[pl.BlockSpec((block_rows, cols), lambda i: (i, 0))],
        out_specs=pl.BlockSpec((block_rows, cols), lambda i: (i, 0)),
        out_shape=jax.ShapeDtypeStruct((rows, cols), x.dtype),
    )(x)


# speedup vs baseline: 31.2470x; 1.0564x over previous
import functools

import jax
import jax.numpy as jnp
from jax.experimental import pallas as pl

_K = 512


def _count_ge(m, t, nsplit):
    cols = m.shape[1]
    w = cols // nsplit
    parts = [
        jnp.sum((m[:, i * w:(i + 1) * w] >= t).astype(jnp.int32), axis=1,
                keepdims=True)
        for i in range(nsplit)
    ]
    tot = parts[0]
    for p in parts[1:]:
        tot = tot + p
    return tot


def _topk_mask_kernel(x_ref, o_ref, *, k, nsplit):
    x = x_ref[:, :]
    b = x.view(jnp.int32)
    m = jnp.where(b < 0, b ^ jnp.int32(0x7FFFFFFF), b)
    cols = m.shape[1]

    cnt0 = _count_ge(m, jnp.zeros((m.shape[0], 1), jnp.int32), nsplit)
    sel = cnt0 >= k
    t = jnp.where(sel, jnp.int32(0), jnp.int32(-(2**31)))
    cur = jnp.where(sel, cnt0, jnp.int32(cols))

    def cond(carry):
        bit, _, cur = carry
        return jnp.logical_and(bit >= 0, jnp.max(cur) > k)

    def body(carry):
        bit, t, cur = carry
        t_try = t + jnp.left_shift(jnp.int32(1), bit)
        cnt = _count_ge(m, t_try, nsplit)
        sel = cnt >= k
        return (bit - 1, jnp.where(sel, t_try, t), jnp.where(sel, cnt, cur))

    _, t, _ = jax.lax.while_loop(cond, body, (jnp.int32(30), t, cur))

    vk = jnp.min(jnp.where(m >= t, m, jnp.int32(2**31 - 1)), axis=1,
                 keepdims=True)
    t_eff = jnp.maximum(vk, jnp.int32(1))
    o_ref[:, :] = jnp.where(m >= t_eff, x, jnp.float32(0.0))


def kernel(x):
    rows, cols = x.shape
    block_rows = 64
    grid = (rows // block_rows,)
    return pl.pallas_call(
        functools.partial(_topk_mask_kernel, k=_K, nsplit=8),
        grid=grid,
        in_specs=[pl.BlockSpec((block_rows, cols), lambda i: (i, 0))],
        out_specs=pl.BlockSpec((block_rows, cols), lambda i: (i, 0)),
        out_shape=jax.ShapeDtypeStruct((rows, cols), x.dtype),
    )(x)


# speedup vs baseline: 32.4149x; 1.0374x over previous
import functools

import jax
import jax.numpy as jnp
from jax.experimental import pallas as pl

_K = 512


def _count_ge(m, t, nsplit):
    cols = m.shape[1]
    w = cols // nsplit
    parts = [
        jnp.sum((m[:, i * w:(i + 1) * w] >= t).astype(jnp.int32), axis=1,
                keepdims=True)
        for i in range(nsplit)
    ]
    tot = parts[0]
    for p in parts[1:]:
        tot = tot + p
    return tot


def _topk_mask_kernel(x_ref, o_ref, *, k, nsplit):
    x = x_ref[:, :]
    b = x.view(jnp.int32)
    m = jnp.where(b < 0, b ^ jnp.int32(0x7FFFFFFF), b)
    cols = m.shape[1]

    cnt0 = _count_ge(m, jnp.zeros((m.shape[0], 1), jnp.int32), nsplit)
    sel = cnt0 >= k
    t = jnp.where(sel, jnp.int32(0), jnp.int32(-(2**31)))
    cur = jnp.where(sel, cnt0, jnp.int32(cols))

    def one_bit(bit, t, cur):
        t_try = t + jnp.left_shift(jnp.int32(1), bit)
        cnt = _count_ge(m, t_try, nsplit)
        sel = cnt >= k
        return jnp.where(sel, t_try, t), jnp.where(sel, cnt, cur)

    t, cur = one_bit(jnp.int32(30), t, cur)

    def cond(carry):
        bit, _, cur = carry
        return jnp.logical_and(bit >= 0, jnp.max(cur) > k)

    def body(carry):
        bit, t, cur = carry
        t, cur = one_bit(bit, t, cur)
        t, cur = one_bit(bit - 1, t, cur)
        return (bit - 2, t, cur)

    _, t, _ = jax.lax.while_loop(cond, body, (jnp.int32(29), t, cur))

    vk = jnp.min(jnp.where(m >= t, m, jnp.int32(2**31 - 1)), axis=1,
                 keepdims=True)
    t_eff = jnp.maximum(vk, jnp.int32(1))
    o_ref[:, :] = jnp.where(m >= t_eff, x, jnp.float32(0.0))


def kernel(x):
    rows, cols = x.shape
    block_rows = 64
    grid = (rows // block_rows,)
    return pl.pallas_call(
        functools.partial(_topk_mask_kernel, k=_K, nsplit=8),
        grid=grid,
        in_specs=[pl.BlockSpec((block_rows, cols), lambda i: (i, 0))],
        out_specs=pl.BlockSpec((block_rows, cols), lambda i: (i, 0)),
        out_shape=jax.ShapeDtypeStruct((rows, cols), x.dtype),
    )(x)
